# Initial kernel scaffold; baseline (speedup 1.0000x reference)
import jax
import jax.numpy as jnp
from jax.experimental import pallas as pl


def kernel(ndata, edge_index, W1, b1, W2, b2, Wres, bres, m):
    raise NotImplementedError("write your pallas kernel here")


# speedup vs baseline: 37.2225x; 37.2225x over previous
import functools
import jax
import jax.numpy as jnp
from jax import lax
from jax.experimental import pallas as pl
from jax.experimental.pallas import tpu as pltpu
from jax.experimental.pallas import tpu_sc as plsc

N = 10000
CH = 128
HCH = 64
NTILES = 16
RPT = 632
NPAD = NTILES * RPT
K = 128
BN = 400


def _sig(x):
    return 1.0 / (1.0 + jnp.exp(-x))


def _tc1_body(nd, w1, b1, w2, b2, wr, br, l_o, r_o, asf_o, bsf_o, nr0_o, xr_o):
    x = nd[...]
    mn = jnp.min(x, axis=1, keepdims=True)
    mx = jnp.max(x, axis=1, keepdims=True)
    xn = (x - mn) / (mx - mn + 1e-5)
    l = jnp.dot(xn, w1[...], preferred_element_type=jnp.float32) + b1[...]
    r = jnp.dot(xn, w2[...], preferred_element_type=jnp.float32) + b2[...]
    xr = jnp.dot(xn, wr[...], preferred_element_type=jnp.float32) + br[...]
    asf = jnp.exp(_sig(2.0 * l))
    bsf = jnp.exp(_sig(2.0 * r))
    l_o[...] = l
    r_o[...] = r
    asf_o[...] = asf
    bsf_o[...] = bsf
    nr0_o[...] = r * asf
    xr_o[...] = xr


def _tc2_body(nr, sl, bsf, ftr_o, fb_o):
    ftr = nr[...] / sl[...]
    ftr_o[...] = ftr
    fb_o[...] = ftr * bsf[...]


def _tc3_body(nl, sr, xr, m0, h_o):
    h_o[...] = (nl[...] / sr[...]) * m0[...] + xr[...]


def _row_spec(width):
    return pl.BlockSpec((BN, width), lambda i: (i, 0))


def _full_spec(shape):
    return pl.BlockSpec(shape, lambda i: tuple(0 for _ in shape))


def _sc_pass_body(src_h, dst_h, tab_s, tab_d, tab_x, init_h, out_h,
                  idx_s, idx_d, idx_so, idx_do, vs, vd, vx, payload,
                  accum, sem, cpt, mul_by_x):
    c = lax.axis_index("c")
    s = lax.axis_index("s")

    pltpu.sync_copy(init_h.at[pl.ds(c * NPAD + s * RPT, RPT)],
                    accum.at[pl.ds(s * RPT, RPT)])
    plsc.subcore_barrier()

    def chunk_body(j, carry):
        base = (s * cpt + j) * K
        pltpu.sync_copy(src_h.at[pl.ds(base, K)], idx_s)
        pltpu.sync_copy(dst_h.at[pl.ds(base, K)], idx_d)

        def off_body(t, carry2):
            sl16 = pl.ds(t * 16, 16)
            idx_so[sl16] = idx_s[sl16] + c * N
            idx_do[sl16] = idx_d[sl16] + c * N
            return carry2

        lax.fori_loop(0, K // 16, off_body, 0)

        h1 = pltpu.async_copy(tab_s.at[idx_so], vs, sem)
        h2 = pltpu.async_copy(tab_d.at[idx_do], vd, sem)
        h3 = pltpu.async_copy(tab_x.at[idx_so], vx, sem)
        h1.wait()
        h2.wait()
        h3.wait()

        def edge_body(k, carry2):
            for g in range(HCH // 16):
                sl16 = pl.ds(g * 16, 16)
                a = jnp.exp(_sig(vs[k, sl16] + vd[k, sl16]))
                payload[k, sl16] = a
                payload[k, pl.ds(HCH + g * 16, 16)] = vx[k, sl16] * a
            return carry2

        lax.fori_loop(0, K, edge_body, 0)
        pltpu.sync_copy(payload, accum.at[idx_d], add=True)
        return carry

    lax.fori_loop(0, cpt, chunk_body, 0)
    plsc.subcore_barrier()
    pltpu.sync_copy(accum.at[pl.ds(s * RPT, RPT)],
                    out_h.at[pl.ds(c * NPAD + s * RPT, RPT)])


def _make_sc_pass(cpt):
    mesh = plsc.VectorSubcoreMesh(core_axis_name="c", subcore_axis_name="s")
    return pl.kernel(
        functools.partial(_sc_pass_body, cpt=cpt, mul_by_x=True),
        out_type=jax.ShapeDtypeStruct((2 * NPAD, CH), jnp.float32),
        mesh=mesh,
        scratch_types=[
            pltpu.VMEM((K,), jnp.int32), pltpu.VMEM((K,), jnp.int32),
            pltpu.VMEM((K,), jnp.int32), pltpu.VMEM((K,), jnp.int32),
            pltpu.VMEM((K, HCH), jnp.float32),
            pltpu.VMEM((K, HCH), jnp.float32),
            pltpu.VMEM((K, HCH), jnp.float32),
            pltpu.VMEM((K, CH), jnp.float32),
            pltpu.VMEM_SHARED((NPAD, CH), jnp.float32),
            pltpu.SemaphoreType.DMA,
        ],
        compiler_params=pltpu.CompilerParams(use_tc_tiling_on_sc=False),
    )


@jax.jit
def kernel(ndata, edge_index, W1, b1, W2, b2, Wres, bres, m):
    n = ndata.shape[0]
    E = edge_index.shape[1]
    grid = (n // BN,)

    l, r, asf, bsf, nr0, xres = pl.pallas_call(
        _tc1_body,
        grid=grid,
        in_specs=[_row_spec(CH), _full_spec((CH, CH)), _full_spec((1, CH)),
                  _full_spec((CH, CH)), _full_spec((1, CH)),
                  _full_spec((CH, CH)), _full_spec((1, CH))],
        out_specs=[_row_spec(CH)] * 6,
        out_shape=[jax.ShapeDtypeStruct((n, CH), jnp.float32)] * 6,
    )(ndata, W1, b1.reshape(1, CH), W2, b2.reshape(1, CH),
      Wres, bres.reshape(1, CH))

    cpt = -(-E // (K * NTILES))
    E_pad = cpt * K * NTILES
    src = edge_index[0]
    dst = edge_index[1]
    srcp = jnp.concatenate([src, jnp.zeros((E_pad - E,), jnp.int32)])
    dstp = jnp.concatenate([dst, jnp.full((E_pad - E,), n, jnp.int32)])

    zpad = jnp.zeros((16, HCH), jnp.float32)

    def halves(x):
        return jnp.concatenate([x[:, :HCH], x[:, HCH:], zpad], axis=0)

    def initp(xa, xb):
        za = jnp.zeros((NPAD - n, CH), jnp.float32)
        ia = jnp.concatenate([xa[:, :HCH], xb[:, :HCH]], axis=1)
        ib = jnp.concatenate([xa[:, HCH:], xb[:, HCH:]], axis=1)
        return jnp.concatenate([ia, za, ib, za], axis=0)

    ltab = halves(l)
    rtab = halves(r)
    init1 = initp(asf, nr0)

    sc_pass = _make_sc_pass(cpt)
    acc1 = sc_pass(srcp, dstp, ltab, ltab, rtab, init1).reshape(2, NPAD, CH)

    s_l = jnp.concatenate([acc1[0, :n, :HCH], acc1[1, :n, :HCH]], axis=1)
    num_r = jnp.concatenate([acc1[0, :n, HCH:], acc1[1, :n, HCH:]], axis=1)

    ftr, fb = pl.pallas_call(
        _tc2_body,
        grid=grid,
        in_specs=[_row_spec(CH)] * 3,
        out_specs=[_row_spec(CH)] * 2,
        out_shape=[jax.ShapeDtypeStruct((n, CH), jnp.float32)] * 2,
    )(num_r, s_l, bsf)

    ftab = halves(ftr)
    init2 = initp(bsf, fb)
    acc2 = sc_pass(srcp, dstp, rtab, rtab, ftab, init2).reshape(2, NPAD, CH)

    s_r = jnp.concatenate([acc2[0, :n, :HCH], acc2[1, :n, :HCH]], axis=1)
    num_l = jnp.concatenate([acc2[0, :n, HCH:], acc2[1, :n, HCH:]], axis=1)

    m0 = m[0].reshape(1, CH)
    h = pl.pallas_call(
        _tc3_body,
        grid=grid,
        in_specs=[_row_spec(CH), _row_spec(CH), _row_spec(CH),
                  _full_spec((1, CH))],
        out_specs=_row_spec(CH),
        out_shape=jax.ShapeDtypeStruct((n, CH), jnp.float32),
    )(num_l, s_r, xres, m0)
    return h


# speedup vs baseline: 116.4815x; 3.1293x over previous
import functools
import jax
import jax.numpy as jnp
from jax import lax
from jax.experimental import pallas as pl
from jax.experimental.pallas import tpu as pltpu
from jax.experimental.pallas import tpu_sc as plsc

N = 10000
CH = 128
HCH = 64
NTILES = 16
RPT = 632
NPAD = NTILES * RPT
K = 128
BN = 400


def _sig(x):
    return 1.0 / (1.0 + jnp.exp(-x))


def _tc1_body(nd, w1, b1, w2, b2, wr, br, l_o, r_o, asf_o, bsf_o, nr0_o, xr_o):
    x = nd[...]
    mn = jnp.min(x, axis=1, keepdims=True)
    mx = jnp.max(x, axis=1, keepdims=True)
    xn = (x - mn) / (mx - mn + 1e-5)
    l = jnp.dot(xn, w1[...], preferred_element_type=jnp.float32) + b1[...]
    r = jnp.dot(xn, w2[...], preferred_element_type=jnp.float32) + b2[...]
    xr = jnp.dot(xn, wr[...], preferred_element_type=jnp.float32) + br[...]
    asf = jnp.exp(_sig(2.0 * l))
    bsf = jnp.exp(_sig(2.0 * r))
    l_o[...] = l
    r_o[...] = r
    asf_o[...] = asf
    bsf_o[...] = bsf
    nr0_o[...] = r * asf
    xr_o[...] = xr


def _tc2_body(nr, sl, bsf, ftr_o, fb_o):
    ftr = nr[...] / sl[...]
    ftr_o[...] = ftr
    fb_o[...] = ftr * bsf[...]


def _tc3_body(nl, sr, xr, m0, h_o):
    h_o[...] = (nl[...] / sr[...]) * m0[...] + xr[...]


def _row_spec(width):
    return pl.BlockSpec((BN, width), lambda i: (i, 0))


def _full_spec(shape):
    return pl.BlockSpec(shape, lambda i: tuple(0 for _ in shape))


def _sc_pass_body(src_h, dst_h, tab_sx, tab_d, init_h, out_h,
                  ixs_a, ixd_a, ixso_a, ixdo_a, ga, gda, pa,
                  ixs_b, ixd_b, ixso_b, ixdo_b, gb, gdb, ixsc,
                  accum, semg, cpt):
    c = lax.axis_index("c")
    s = lax.axis_index("s")
    coff = c * N

    pltpu.sync_copy(init_h.at[pl.ds(c * NPAD + s * RPT, RPT)],
                    accum.at[pl.ds(s * RPT, RPT)])
    plsc.subcore_barrier()

    def load_idx(base, ixs, ixd, ixso, ixdo):
        pltpu.sync_copy(src_h.at[pl.ds(base, K)], ixs)
        pltpu.sync_copy(dst_h.at[pl.ds(base, K)], ixd)

        @plsc.parallel_loop(0, K // 16, unroll=8)
        def off_body(t):
            sl16 = pl.ds(t * 16, 16)
            ixso[sl16] = ixs[sl16] + coff
            ixdo[sl16] = ixd[sl16] + coff

    def fire_gather(ixso, ixdo, g, gd):
        pltpu.async_copy(tab_sx.at[ixso], g, semg)
        pltpu.async_copy(tab_d.at[ixdo], gd, semg)

    def wait_gather(ixso, ixdo, g, gd):
        pltpu.make_async_copy(tab_sx.at[ixso], g, semg).wait()
        pltpu.make_async_copy(tab_d.at[ixdo], gd, semg).wait()

    def compute(g, gd, p):
        @plsc.parallel_loop(0, K, unroll=4)
        def edge_body(k):
            for gi in range(HCH // 16):
                sl16 = pl.ds(gi * 16, 16)
                slx = pl.ds(HCH + gi * 16, 16)
                a = jnp.exp(_sig(g[k, sl16] + gd[k, sl16]))
                p[k, sl16] = a
                p[k, slx] = g[k, slx] * a

    def snap_idx(ixd):
        @plsc.parallel_loop(0, K // 16, unroll=8)
        def cp_body(t):
            sl16 = pl.ds(t * 16, 16)
            ixsc[sl16] = ixd[sl16]

    def chunk_body(j, carry):
        base = (s * cpt + j) * K
        load_idx(base, ixs_a, ixd_a, ixso_a, ixdo_a)
        fire_gather(ixso_a, ixdo_a, ga, gda)
        wait_gather(ixso_a, ixdo_a, ga, gda)
        compute(ga, gda, pa)
        pltpu.sync_copy(pa, accum.at[ixd_a], add=True)
        return carry

    lax.fori_loop(0, cpt, chunk_body, 0)

    plsc.subcore_barrier()
    pltpu.sync_copy(accum.at[pl.ds(s * RPT, RPT)],
                    out_h.at[pl.ds(c * NPAD + s * RPT, RPT)])


def _make_sc_pass(cpt):
    mesh = plsc.VectorSubcoreMesh(core_axis_name="c", subcore_axis_name="s")
    ibuf = pltpu.VMEM((K,), jnp.int32)
    return pl.kernel(
        functools.partial(_sc_pass_body, cpt=cpt),
        out_type=jax.ShapeDtypeStruct((2 * NPAD, CH), jnp.float32),
        mesh=mesh,
        scratch_types=[
            ibuf, ibuf, ibuf, ibuf,
            pltpu.VMEM((K, CH), jnp.float32),
            pltpu.VMEM((K, HCH), jnp.float32),
            pltpu.VMEM((K, CH), jnp.float32),
            ibuf, ibuf, ibuf, ibuf,
            pltpu.VMEM((K, CH), jnp.float32),
            pltpu.VMEM((K, HCH), jnp.float32),
            ibuf,
            pltpu.VMEM_SHARED((NPAD, CH), jnp.float32),
            pltpu.SemaphoreType.DMA,
        ],
        compiler_params=pltpu.CompilerParams(use_tc_tiling_on_sc=False),
    )


@jax.jit
def kernel(ndata, edge_index, W1, b1, W2, b2, Wres, bres, m):
    n = ndata.shape[0]
    E = edge_index.shape[1]
    grid = (n // BN,)

    l, r, asf, bsf, nr0, xres = pl.pallas_call(
        _tc1_body,
        grid=grid,
        in_specs=[_row_spec(CH), _full_spec((CH, CH)), _full_spec((1, CH)),
                  _full_spec((CH, CH)), _full_spec((1, CH)),
                  _full_spec((CH, CH)), _full_spec((1, CH))],
        out_specs=[_row_spec(CH)] * 6,
        out_shape=[jax.ShapeDtypeStruct((n, CH), jnp.float32)] * 6,
    )(ndata, W1, b1.reshape(1, CH), W2, b2.reshape(1, CH),
      Wres, bres.reshape(1, CH))

    cpt = 2 * (-(-E // (K * NTILES * 2)))
    E_pad = cpt * K * NTILES
    src = edge_index[0]
    dst = edge_index[1]
    srcp = jnp.concatenate([src, jnp.zeros((E_pad + K - E,), jnp.int32)])
    dstp = jnp.concatenate([dst, jnp.full((E_pad + K - E,), n, jnp.int32)])

    zpad128 = jnp.zeros((16, CH), jnp.float32)
    zpad64 = jnp.zeros((16, HCH), jnp.float32)

    def pair_tab(u, v):
        ta = jnp.concatenate([u[:, :HCH], v[:, :HCH]], axis=1)
        tb = jnp.concatenate([u[:, HCH:], v[:, HCH:]], axis=1)
        return jnp.concatenate([ta, tb, zpad128], axis=0)

    def halves(x):
        return jnp.concatenate([x[:, :HCH], x[:, HCH:], zpad64], axis=0)

    def initp(xa, xb):
        za = jnp.zeros((NPAD - n, CH), jnp.float32)
        ia = jnp.concatenate([xa[:, :HCH], xb[:, :HCH]], axis=1)
        ib = jnp.concatenate([xa[:, HCH:], xb[:, HCH:]], axis=1)
        return jnp.concatenate([ia, za, ib, za], axis=0)

    ltab = halves(l)
    rtab = halves(r)

    sc_pass = _make_sc_pass(cpt)
    acc1 = sc_pass(srcp, dstp, pair_tab(l, r), ltab,
                   initp(asf, nr0)).reshape(2, NPAD, CH)

    s_l = jnp.concatenate([acc1[0, :n, :HCH], acc1[1, :n, :HCH]], axis=1)
    num_r = jnp.concatenate([acc1[0, :n, HCH:], acc1[1, :n, HCH:]], axis=1)

    ftr, fb = pl.pallas_call(
        _tc2_body,
        grid=grid,
        in_specs=[_row_spec(CH)] * 3,
        out_specs=[_row_spec(CH)] * 2,
        out_shape=[jax.ShapeDtypeStruct((n, CH), jnp.float32)] * 2,
    )(num_r, s_l, bsf)

    acc2 = sc_pass(srcp, dstp, pair_tab(r, ftr), rtab,
                   initp(bsf, fb)).reshape(2, NPAD, CH)

    s_r = jnp.concatenate([acc2[0, :n, :HCH], acc2[1, :n, :HCH]], axis=1)
    num_l = jnp.concatenate([acc2[0, :n, HCH:], acc2[1, :n, HCH:]], axis=1)

    m0 = m[0].reshape(1, CH)
    h = pl.pallas_call(
        _tc3_body,
        grid=grid,
        in_specs=[_row_spec(CH), _row_spec(CH), _row_spec(CH),
                  _full_spec((1, CH))],
        out_specs=_row_spec(CH),
        out_shape=jax.ShapeDtypeStruct((n, CH), jnp.float32),
    )(num_l, s_r, xres, m0)
    return h


# speedup vs baseline: 165.2926x; 1.4190x over previous
import functools
import jax
import jax.numpy as jnp
from jax import lax
from jax.experimental import pallas as pl
from jax.experimental.pallas import tpu as pltpu
from jax.experimental.pallas import tpu_sc as plsc

N = 10000
CH = 128
HCH = 64
NTILES = 16
RPT = 632
NPAD = NTILES * RPT
K = 96
BN = 400


def _sig(x):
    return 1.0 / (1.0 + jnp.exp(-x))


def _tc1_body(nd, w1, b1, w2, b2, wr, br, l_o, r_o, asf_o, bsf_o, nr0_o, xr_o):
    x = nd[...]
    mn = jnp.min(x, axis=1, keepdims=True)
    mx = jnp.max(x, axis=1, keepdims=True)
    xn = (x - mn) / (mx - mn + 1e-5)
    l = jnp.dot(xn, w1[...], preferred_element_type=jnp.float32) + b1[...]
    r = jnp.dot(xn, w2[...], preferred_element_type=jnp.float32) + b2[...]
    xr = jnp.dot(xn, wr[...], preferred_element_type=jnp.float32) + br[...]
    asf = jnp.exp(_sig(2.0 * l))
    bsf = jnp.exp(_sig(2.0 * r))
    l_o[...] = l
    r_o[...] = r
    asf_o[...] = asf
    bsf_o[...] = bsf
    nr0_o[...] = r * asf
    xr_o[...] = xr


def _tc2_body(nr, sl, bsf, ftr_o, fb_o):
    ftr = nr[...] / sl[...]
    ftr_o[...] = ftr
    fb_o[...] = ftr * bsf[...]


def _tc3_body(nl, sr, xr, m0, h_o):
    h_o[...] = (nl[...] / sr[...]) * m0[...] + xr[...]


def _row_spec(width):
    return pl.BlockSpec((BN, width), lambda i: (i, 0))


def _full_spec(shape):
    return pl.BlockSpec(shape, lambda i: tuple(0 for _ in shape))


def _sc_pass_body(src_h, dst_h, tab_s, tab_d, tab_x, init_h, out_h,
                  ixs_a, ixd_a, ixso_a, ixdo_a, vs_a, vd_a, vx_a, pa,
                  ixs_b, ixd_b, ixso_b, ixdo_b, vs_b, vd_b, vx_b, ixsc,
                  accum, semg, cpt):
    c = lax.axis_index("c")
    s = lax.axis_index("s")
    coff = c * N

    pltpu.sync_copy(init_h.at[pl.ds(c * NPAD + s * RPT, RPT)],
                    accum.at[pl.ds(s * RPT, RPT)])
    plsc.subcore_barrier()

    def load_idx(base, ixs, ixd, ixso, ixdo):
        pltpu.sync_copy(src_h.at[pl.ds(base, K)], ixs)
        pltpu.sync_copy(dst_h.at[pl.ds(base, K)], ixd)

        @plsc.parallel_loop(0, K // 16, unroll=8)
        def off_body(t):
            sl16 = pl.ds(t * 16, 16)
            ixso[sl16] = ixs[sl16] + coff
            ixdo[sl16] = ixd[sl16] + coff

    def fire_gather(ixso, ixdo, vs, vd, vx):
        pltpu.async_copy(tab_s.at[ixso], vs, semg)
        pltpu.async_copy(tab_d.at[ixdo], vd, semg)
        pltpu.async_copy(tab_x.at[ixso], vx, semg)

    def wait_gather(ixso, ixdo, vs, vd, vx):
        pltpu.make_async_copy(tab_s.at[ixso], vs, semg).wait()
        pltpu.make_async_copy(tab_d.at[ixdo], vd, semg).wait()
        pltpu.make_async_copy(tab_x.at[ixso], vx, semg).wait()

    def compute(vs, vd, vx):
        @plsc.parallel_loop(0, K, unroll=4)
        def edge_body(k):
            for gi in range(HCH // 16):
                sl16 = pl.ds(gi * 16, 16)
                a = jnp.exp(_sig(vs[k, sl16] + vd[k, sl16]))
                pa[k, sl16] = a
                pa[k, pl.ds(HCH + gi * 16, 16)] = vx[k, sl16] * a

    def snap_idx(ixd):
        @plsc.parallel_loop(0, K // 16, unroll=8)
        def cp_body(t):
            sl16 = pl.ds(t * 16, 16)
            ixsc[sl16] = ixd[sl16]

    base0 = s * cpt * K
    load_idx(base0, ixs_a, ixd_a, ixso_a, ixdo_a)
    fire_gather(ixso_a, ixdo_a, vs_a, vd_a, vx_a)

    def chunk_body(j, carry):
        nbase = (s * cpt + j + 1) * K

        @pl.when(jnp.bitwise_and(j, 1) == 0)
        def _():
            wait_gather(ixso_a, ixdo_a, vs_a, vd_a, vx_a)
            load_idx(nbase, ixs_b, ixd_b, ixso_b, ixdo_b)
            fire_gather(ixso_b, ixdo_b, vs_b, vd_b, vx_b)
            compute(vs_a, vd_a, vx_a)
            snap_idx(ixd_a)

        @pl.when(jnp.bitwise_and(j, 1) == 1)
        def _():
            wait_gather(ixso_b, ixdo_b, vs_b, vd_b, vx_b)
            load_idx(nbase, ixs_a, ixd_a, ixso_a, ixdo_a)
            fire_gather(ixso_a, ixdo_a, vs_a, vd_a, vx_a)
            compute(vs_b, vd_b, vx_b)
            snap_idx(ixd_b)

        pltpu.sync_copy(pa, accum.at[ixsc], add=True)
        return carry

    lax.fori_loop(0, cpt, chunk_body, 0)

    wait_gather(ixso_a, ixdo_a, vs_a, vd_a, vx_a)

    plsc.subcore_barrier()
    pltpu.sync_copy(accum.at[pl.ds(s * RPT, RPT)],
                    out_h.at[pl.ds(c * NPAD + s * RPT, RPT)])


def _make_sc_pass(cpt):
    mesh = plsc.VectorSubcoreMesh(core_axis_name="c", subcore_axis_name="s")
    ibuf = pltpu.VMEM((K,), jnp.int32)
    hbuf = pltpu.VMEM((K, HCH), jnp.float32)
    return pl.kernel(
        functools.partial(_sc_pass_body, cpt=cpt),
        out_type=jax.ShapeDtypeStruct((2 * NPAD, CH), jnp.float32),
        mesh=mesh,
        scratch_types=[
            ibuf, ibuf, ibuf, ibuf, hbuf, hbuf, hbuf,
            pltpu.VMEM((K, CH), jnp.float32),
            ibuf, ibuf, ibuf, ibuf, hbuf, hbuf, hbuf,
            ibuf,
            pltpu.VMEM_SHARED((NPAD, CH), jnp.float32),
            pltpu.SemaphoreType.DMA,
        ],
        compiler_params=pltpu.CompilerParams(use_tc_tiling_on_sc=False),
    )


@jax.jit
def kernel(ndata, edge_index, W1, b1, W2, b2, Wres, bres, m):
    n = ndata.shape[0]
    E = edge_index.shape[1]
    grid = (n // BN,)

    l, r, asf, bsf, nr0, xres = pl.pallas_call(
        _tc1_body,
        grid=grid,
        in_specs=[_row_spec(CH), _full_spec((CH, CH)), _full_spec((1, CH)),
                  _full_spec((CH, CH)), _full_spec((1, CH)),
                  _full_spec((CH, CH)), _full_spec((1, CH))],
        out_specs=[_row_spec(CH)] * 6,
        out_shape=[jax.ShapeDtypeStruct((n, CH), jnp.float32)] * 6,
    )(ndata, W1, b1.reshape(1, CH), W2, b2.reshape(1, CH),
      Wres, bres.reshape(1, CH))

    cpt = 2 * (-(-E // (K * NTILES * 2)))
    E_pad = cpt * K * NTILES
    src = edge_index[0]
    dst = edge_index[1]
    srcp = jnp.concatenate([src, jnp.zeros((E_pad + K - E,), jnp.int32)])
    dstp = jnp.concatenate([dst, jnp.full((E_pad + K - E,), n, jnp.int32)])

    zpad64 = jnp.zeros((16, HCH), jnp.float32)

    def halves(x):
        return jnp.concatenate([x[:, :HCH], x[:, HCH:], zpad64], axis=0)

    def initp(xa, xb):
        za = jnp.zeros((NPAD - n, CH), jnp.float32)
        ia = jnp.concatenate([xa[:, :HCH], xb[:, :HCH]], axis=1)
        ib = jnp.concatenate([xa[:, HCH:], xb[:, HCH:]], axis=1)
        return jnp.concatenate([ia, za, ib, za], axis=0)

    ltab = halves(l)
    rtab = halves(r)

    sc_pass = _make_sc_pass(cpt)
    acc1 = sc_pass(srcp, dstp, ltab, ltab, rtab,
                   initp(asf, nr0)).reshape(2, NPAD, CH)

    s_l = jnp.concatenate([acc1[0, :n, :HCH], acc1[1, :n, :HCH]], axis=1)
    num_r = jnp.concatenate([acc1[0, :n, HCH:], acc1[1, :n, HCH:]], axis=1)

    ftr, fb = pl.pallas_call(
        _tc2_body,
        grid=grid,
        in_specs=[_row_spec(CH)] * 3,
        out_specs=[_row_spec(CH)] * 2,
        out_shape=[jax.ShapeDtypeStruct((n, CH), jnp.float32)] * 2,
    )(num_r, s_l, bsf)

    acc2 = sc_pass(srcp, dstp, rtab, rtab, halves(ftr),
                   initp(bsf, fb)).reshape(2, NPAD, CH)

    s_r = jnp.concatenate([acc2[0, :n, :HCH], acc2[1, :n, :HCH]], axis=1)
    num_l = jnp.concatenate([acc2[0, :n, HCH:], acc2[1, :n, HCH:]], axis=1)

    m0 = m[0].reshape(1, CH)
    h = pl.pallas_call(
        _tc3_body,
        grid=grid,
        in_specs=[_row_spec(CH), _row_spec(CH), _row_spec(CH),
                  _full_spec((1, CH))],
        out_specs=_row_spec(CH),
        out_shape=jax.ShapeDtypeStruct((n, CH), jnp.float32),
    )(num_l, s_r, xres, m0)
    return h


# speedup vs baseline: 172.0789x; 1.0411x over previous
import functools
import jax
import jax.numpy as jnp
from jax import lax
from jax.experimental import pallas as pl
from jax.experimental.pallas import tpu as pltpu
from jax.experimental.pallas import tpu_sc as plsc

N = 10000
CH = 128
HCH = 64
NTILES = 16
RPT = 632
NPAD = NTILES * RPT
K = 96
BN = 400
SUBS = ((0, 96), (96, 96), (192, 96), (288, 96), (384, 96), (480, 96),
        (576, 56))


def _sig(x):
    return 1.0 / (1.0 + jnp.exp(-x))


def _tc1_body(nd, w1, b1, w2, b2, wr, br, l_o, r_o, xr_o):
    x = nd[...]
    mn = jnp.min(x, axis=1, keepdims=True)
    mx = jnp.max(x, axis=1, keepdims=True)
    xn = (x - mn) / (mx - mn + 1e-5)
    l_o[...] = jnp.dot(xn, w1[...], preferred_element_type=jnp.float32) + b1[...]
    r_o[...] = jnp.dot(xn, w2[...], preferred_element_type=jnp.float32) + b2[...]
    xr_o[...] = jnp.dot(xn, wr[...], preferred_element_type=jnp.float32) + br[...]


def _row_spec(width):
    return pl.BlockSpec((BN, width), lambda i: (i, 0))


def _full_spec(shape):
    return pl.BlockSpec(shape, lambda i: tuple(0 for _ in shape))


def _sc_pass_body(src_h, dst_h, tab_s, tab_d, tab_x, xadd_h, m_h, out_h,
                  ixs_a, ixd_a, ixso_a, ixdo_a, vs_a, vd_a, vx_a, pa,
                  ixs_b, ixd_b, ixso_b, ixdo_b, vs_b, vd_b, vx_b, ixsc,
                  accum, semg, cpt):
    c = lax.axis_index("c")
    s = lax.axis_index("s")
    coff = c * NPAD
    rbase = s * RPT

    for off, sz in SUBS:
        pltpu.sync_copy(tab_s.at[pl.ds(coff + rbase + off, sz)],
                        vs_a.at[pl.ds(0, sz)])
        pltpu.sync_copy(tab_x.at[pl.ds(coff + rbase + off, sz)],
                        vx_a.at[pl.ds(0, sz)])

        @plsc.parallel_loop(0, sz, unroll=4)
        def pro_body(k):
            for gi in range(HCH // 16):
                sl16 = pl.ds(gi * 16, 16)
                e = jnp.exp(_sig(2.0 * vs_a[k, sl16]))
                pa[k, sl16] = e
                pa[k, pl.ds(HCH + gi * 16, 16)] = vx_a[k, sl16] * e

        pltpu.sync_copy(pa.at[pl.ds(0, sz)],
                        accum.at[pl.ds(rbase + off, sz)])

    plsc.subcore_barrier()

    def load_idx(base, ixs, ixd, ixso, ixdo):
        pltpu.sync_copy(src_h.at[pl.ds(base, K)], ixs)
        pltpu.sync_copy(dst_h.at[pl.ds(base, K)], ixd)

        @plsc.parallel_loop(0, K // 16, unroll=6)
        def off_body(t):
            sl16 = pl.ds(t * 16, 16)
            ixso[sl16] = ixs[sl16] + coff
            ixdo[sl16] = ixd[sl16] + coff

    def fire_gather(ixso, ixdo, vs, vd, vx):
        pltpu.async_copy(tab_s.at[ixso], vs, semg)
        pltpu.async_copy(tab_d.at[ixdo], vd, semg)
        pltpu.async_copy(tab_x.at[ixso], vx, semg)

    def wait_gather(ixso, ixdo, vs, vd, vx):
        pltpu.make_async_copy(tab_s.at[ixso], vs, semg).wait()
        pltpu.make_async_copy(tab_d.at[ixdo], vd, semg).wait()
        pltpu.make_async_copy(tab_x.at[ixso], vx, semg).wait()

    def compute(vs, vd, vx):
        @plsc.parallel_loop(0, K, unroll=4)
        def edge_body(k):
            for gi in range(HCH // 16):
                sl16 = pl.ds(gi * 16, 16)
                a = jnp.exp(_sig(vs[k, sl16] + vd[k, sl16]))
                pa[k, sl16] = a
                pa[k, pl.ds(HCH + gi * 16, 16)] = vx[k, sl16] * a

    def snap_idx(ixd):
        @plsc.parallel_loop(0, K // 16, unroll=6)
        def cp_body(t):
            sl16 = pl.ds(t * 16, 16)
            ixsc[sl16] = ixd[sl16]

    base0 = s * cpt * K
    load_idx(base0, ixs_a, ixd_a, ixso_a, ixdo_a)
    fire_gather(ixso_a, ixdo_a, vs_a, vd_a, vx_a)

    def chunk_body(j, carry):
        nbase = (s * cpt + j + 1) * K

        @pl.when(jnp.bitwise_and(j, 1) == 0)
        def _():
            wait_gather(ixso_a, ixdo_a, vs_a, vd_a, vx_a)
            load_idx(nbase, ixs_b, ixd_b, ixso_b, ixdo_b)
            fire_gather(ixso_b, ixdo_b, vs_b, vd_b, vx_b)
            compute(vs_a, vd_a, vx_a)
            snap_idx(ixd_a)

        @pl.when(jnp.bitwise_and(j, 1) == 1)
        def _():
            wait_gather(ixso_b, ixdo_b, vs_b, vd_b, vx_b)
            load_idx(nbase, ixs_a, ixd_a, ixso_a, ixdo_a)
            fire_gather(ixso_a, ixdo_a, vs_a, vd_a, vx_a)
            compute(vs_b, vd_b, vx_b)
            snap_idx(ixd_b)

        pltpu.sync_copy(pa, accum.at[ixsc], add=True)
        return carry

    lax.fori_loop(0, cpt, chunk_body, 0)

    wait_gather(ixso_a, ixdo_a, vs_a, vd_a, vx_a)
    plsc.subcore_barrier()

    pltpu.sync_copy(m_h.at[pl.ds(c * HCH, HCH)], vd_a.at[0])
    for off, sz in SUBS:
        pltpu.sync_copy(accum.at[pl.ds(rbase + off, sz)],
                        pa.at[pl.ds(0, sz)])
        pltpu.sync_copy(xadd_h.at[pl.ds(coff + rbase + off, sz)],
                        vx_a.at[pl.ds(0, sz)])

        @plsc.parallel_loop(0, sz, unroll=4)
        def epi_body(k):
            for gi in range(HCH // 16):
                sl16 = pl.ds(gi * 16, 16)
                num = pa[k, pl.ds(HCH + gi * 16, 16)]
                vs_a[k, sl16] = ((num / pa[k, sl16]) * vd_a[0, sl16]
                                 + vx_a[k, sl16])

        pltpu.sync_copy(vs_a.at[pl.ds(0, sz)],
                        out_h.at[pl.ds(coff + rbase + off, sz)])


def _make_sc_pass(cpt):
    mesh = plsc.VectorSubcoreMesh(core_axis_name="c", subcore_axis_name="s")
    ibuf = pltpu.VMEM((K,), jnp.int32)
    hbuf = pltpu.VMEM((K, HCH), jnp.float32)
    return pl.kernel(
        functools.partial(_sc_pass_body, cpt=cpt),
        out_type=jax.ShapeDtypeStruct((2 * NPAD, HCH), jnp.float32),
        mesh=mesh,
        scratch_types=[
            ibuf, ibuf, ibuf, ibuf, hbuf, hbuf, hbuf,
            pltpu.VMEM((K, CH), jnp.float32),
            ibuf, ibuf, ibuf, ibuf, hbuf, hbuf, hbuf,
            ibuf,
            pltpu.VMEM_SHARED((NPAD, CH), jnp.float32),
            pltpu.SemaphoreType.DMA,
        ],
        compiler_params=pltpu.CompilerParams(use_tc_tiling_on_sc=False),
    )


@jax.jit
def kernel(ndata, edge_index, W1, b1, W2, b2, Wres, bres, m):
    n = ndata.shape[0]
    E = edge_index.shape[1]
    grid = (n // BN,)

    l, r, xres = pl.pallas_call(
        _tc1_body,
        grid=grid,
        in_specs=[_row_spec(CH), _full_spec((CH, CH)), _full_spec((1, CH)),
                  _full_spec((CH, CH)), _full_spec((1, CH)),
                  _full_spec((CH, CH)), _full_spec((1, CH))],
        out_specs=[_row_spec(CH)] * 3,
        out_shape=[jax.ShapeDtypeStruct((n, CH), jnp.float32)] * 3,
    )(ndata, W1, b1.reshape(1, CH), W2, b2.reshape(1, CH),
      Wres, bres.reshape(1, CH))

    cpt = 2 * (-(-E // (K * NTILES * 2)))
    E_pad = cpt * K * NTILES
    src = edge_index[0]
    dst = edge_index[1]
    srcp = jnp.concatenate([src, jnp.zeros((E_pad + K - E,), jnp.int32)])
    dstp = jnp.concatenate([dst, jnp.full((E_pad + K - E,), n, jnp.int32)])

    znp = jnp.zeros((NPAD - n, HCH), jnp.float32)

    def padhalves(x):
        return jnp.concatenate([x[:, :HCH], znp, x[:, HCH:], znp], axis=0)

    ltab = padhalves(l)
    rtab = padhalves(r)
    xtab = padhalves(xres)
    ztab = jnp.zeros((2 * NPAD, HCH), jnp.float32)
    ones = jnp.ones((CH,), jnp.float32)
    m0 = m[0].reshape(CH)

    sc_pass = _make_sc_pass(cpt)
    ftab = sc_pass(srcp, dstp, ltab, ltab, rtab, ztab, ones)
    hh = sc_pass(srcp, dstp, rtab, rtab, ftab, xtab, m0)
    return jnp.concatenate([hh[:n], hh[NPAD:NPAD + n]], axis=1)


# speedup vs baseline: 172.6625x; 1.0034x over previous
import functools
import jax
import jax.numpy as jnp
from jax import lax
from jax.experimental import pallas as pl
from jax.experimental.pallas import tpu as pltpu
from jax.experimental.pallas import tpu_sc as plsc

N = 10000
CH = 128
HCH = 64
NTILES = 16
RPT = 632
NPAD = NTILES * RPT
K = 96
BN = 400
SUBS = ((0, 96), (96, 96), (192, 96), (288, 96), (384, 96), (480, 96),
        (576, 56))


def _sig(x):
    return 1.0 / (1.0 + jnp.exp(-x))


def _tc1_body(nd, w1, b1, w2, b2, wr, br, l_o, r_o, xr_o):
    x = nd[...]
    mn = jnp.min(x, axis=1, keepdims=True)
    mx = jnp.max(x, axis=1, keepdims=True)
    xn = (x - mn) / (mx - mn + 1e-5)
    l_o[...] = jnp.dot(xn, w1[...], preferred_element_type=jnp.float32) + b1[...]
    r_o[...] = jnp.dot(xn, w2[...], preferred_element_type=jnp.float32) + b2[...]
    xr_o[...] = jnp.dot(xn, wr[...], preferred_element_type=jnp.float32) + br[...]


def _row_spec(width):
    return pl.BlockSpec((BN, width), lambda i: (i, 0))


def _full_spec(shape):
    return pl.BlockSpec(shape, lambda i: tuple(0 for _ in shape))


def _sc_pass_body(src_h, dst_h, tab_s, tab_d, tab_x, xadd_h, m_h, out_h,
                  ixs_a, ixd_a, ixso_a, ixdo_a, vs_a, vd_a, vx_a, pa,
                  ixs_b, ixd_b, ixso_b, ixdo_b, vs_b, vd_b, vx_b, ixsc,
                  accum, semg, cpt):
    c = lax.axis_index("c")
    s = lax.axis_index("s")
    coff = c * NPAD
    rbase = s * RPT

    for off, sz in SUBS:
        pltpu.sync_copy(tab_s.at[pl.ds(coff + rbase + off, sz)],
                        vs_a.at[pl.ds(0, sz)])
        pltpu.sync_copy(tab_x.at[pl.ds(coff + rbase + off, sz)],
                        vx_a.at[pl.ds(0, sz)])

        @plsc.parallel_loop(0, sz, unroll=4)
        def pro_body(k):
            for gi in range(HCH // 16):
                sl16 = pl.ds(gi * 16, 16)
                e = jnp.exp(_sig(2.0 * vs_a[k, sl16]))
                pa[k, sl16] = e
                pa[k, pl.ds(HCH + gi * 16, 16)] = vx_a[k, sl16] * e

        pltpu.sync_copy(pa.at[pl.ds(0, sz)],
                        accum.at[pl.ds(rbase + off, sz)])

    plsc.subcore_barrier()

    def load_idx(base, ixs, ixd, ixso, ixdo):
        pltpu.sync_copy(src_h.at[pl.ds(base, K)], ixs)
        pltpu.sync_copy(dst_h.at[pl.ds(base, K)], ixd)

        @plsc.parallel_loop(0, K // 16, unroll=6)
        def off_body(t):
            sl16 = pl.ds(t * 16, 16)
            ixso[sl16] = ixs[sl16] + coff
            ixdo[sl16] = ixd[sl16] + coff

    def fire_gather(ixso, ixdo, vs, vd, vx):
        pltpu.async_copy(tab_s.at[ixso], vs, semg)
        pltpu.async_copy(tab_d.at[ixdo], vd, semg)
        pltpu.async_copy(tab_x.at[ixso], vx, semg)

    def wait_gather(ixso, ixdo, vs, vd, vx):
        pltpu.make_async_copy(tab_s.at[ixso], vs, semg).wait()
        pltpu.make_async_copy(tab_d.at[ixdo], vd, semg).wait()
        pltpu.make_async_copy(tab_x.at[ixso], vx, semg).wait()

    def compute(vs, vd, vx):
        @plsc.parallel_loop(0, K, unroll=8)
        def edge_body(k):
            for gi in range(HCH // 16):
                sl16 = pl.ds(gi * 16, 16)
                a = jnp.exp(_sig(vs[k, sl16] + vd[k, sl16]))
                pa[k, sl16] = a
                pa[k, pl.ds(HCH + gi * 16, 16)] = vx[k, sl16] * a

    def snap_idx(ixd):
        @plsc.parallel_loop(0, K // 16, unroll=6)
        def cp_body(t):
            sl16 = pl.ds(t * 16, 16)
            ixsc[sl16] = ixd[sl16]

    base0 = s * cpt * K
    load_idx(base0, ixs_a, ixd_a, ixso_a, ixdo_a)
    fire_gather(ixso_a, ixdo_a, vs_a, vd_a, vx_a)

    def chunk_body(j, carry):
        nbase = (s * cpt + j + 1) * K

        @pl.when(jnp.bitwise_and(j, 1) == 0)
        def _():
            wait_gather(ixso_a, ixdo_a, vs_a, vd_a, vx_a)
            load_idx(nbase, ixs_b, ixd_b, ixso_b, ixdo_b)
            fire_gather(ixso_b, ixdo_b, vs_b, vd_b, vx_b)
            compute(vs_a, vd_a, vx_a)
            snap_idx(ixd_a)

        @pl.when(jnp.bitwise_and(j, 1) == 1)
        def _():
            wait_gather(ixso_b, ixdo_b, vs_b, vd_b, vx_b)
            load_idx(nbase, ixs_a, ixd_a, ixso_a, ixdo_a)
            fire_gather(ixso_a, ixdo_a, vs_a, vd_a, vx_a)
            compute(vs_b, vd_b, vx_b)
            snap_idx(ixd_b)

        pltpu.sync_copy(pa, accum.at[ixsc], add=True)
        return carry

    lax.fori_loop(0, cpt, chunk_body, 0)

    wait_gather(ixso_a, ixdo_a, vs_a, vd_a, vx_a)
    plsc.subcore_barrier()

    pltpu.sync_copy(m_h.at[pl.ds(c * HCH, HCH)], vd_a.at[0])
    for off, sz in SUBS:
        pltpu.sync_copy(accum.at[pl.ds(rbase + off, sz)],
                        pa.at[pl.ds(0, sz)])
        pltpu.sync_copy(xadd_h.at[pl.ds(coff + rbase + off, sz)],
                        vx_a.at[pl.ds(0, sz)])

        @plsc.parallel_loop(0, sz, unroll=4)
        def epi_body(k):
            for gi in range(HCH // 16):
                sl16 = pl.ds(gi * 16, 16)
                num = pa[k, pl.ds(HCH + gi * 16, 16)]
                vs_a[k, sl16] = ((num / pa[k, sl16]) * vd_a[0, sl16]
                                 + vx_a[k, sl16])

        pltpu.sync_copy(vs_a.at[pl.ds(0, sz)],
                        out_h.at[pl.ds(coff + rbase + off, sz)])


def _make_sc_pass(cpt):
    mesh = plsc.VectorSubcoreMesh(core_axis_name="c", subcore_axis_name="s")
    ibuf = pltpu.VMEM((K,), jnp.int32)
    hbuf = pltpu.VMEM((K, HCH), jnp.float32)
    return pl.kernel(
        functools.partial(_sc_pass_body, cpt=cpt),
        out_type=jax.ShapeDtypeStruct((2 * NPAD, HCH), jnp.float32),
        mesh=mesh,
        scratch_types=[
            ibuf, ibuf, ibuf, ibuf, hbuf, hbuf, hbuf,
            pltpu.VMEM((K, CH), jnp.float32),
            ibuf, ibuf, ibuf, ibuf, hbuf, hbuf, hbuf,
            ibuf,
            pltpu.VMEM_SHARED((NPAD, CH), jnp.float32),
            pltpu.SemaphoreType.DMA,
        ],
        compiler_params=pltpu.CompilerParams(use_tc_tiling_on_sc=False),
    )


@jax.jit
def kernel(ndata, edge_index, W1, b1, W2, b2, Wres, bres, m):
    n = ndata.shape[0]
    E = edge_index.shape[1]
    grid = (n // BN,)

    l, r, xres = pl.pallas_call(
        _tc1_body,
        grid=grid,
        in_specs=[_row_spec(CH), _full_spec((CH, CH)), _full_spec((1, CH)),
                  _full_spec((CH, CH)), _full_spec((1, CH)),
                  _full_spec((CH, CH)), _full_spec((1, CH))],
        out_specs=[_row_spec(CH)] * 3,
        out_shape=[jax.ShapeDtypeStruct((n, CH), jnp.float32)] * 3,
    )(ndata, W1, b1.reshape(1, CH), W2, b2.reshape(1, CH),
      Wres, bres.reshape(1, CH))

    cpt = 2 * (-(-E // (K * NTILES * 2)))
    E_pad = cpt * K * NTILES
    src = edge_index[0]
    dst = edge_index[1]
    srcp = jnp.concatenate([src, jnp.zeros((E_pad + K - E,), jnp.int32)])
    dstp = jnp.concatenate([dst, jnp.full((E_pad + K - E,), n, jnp.int32)])

    znp = jnp.zeros((NPAD - n, HCH), jnp.float32)

    def padhalves(x):
        return jnp.concatenate([x[:, :HCH], znp, x[:, HCH:], znp], axis=0)

    ltab = padhalves(l)
    rtab = padhalves(r)
    xtab = padhalves(xres)
    ztab = jnp.zeros((2 * NPAD, HCH), jnp.float32)
    ones = jnp.ones((CH,), jnp.float32)
    m0 = m[0].reshape(CH)

    sc_pass = _make_sc_pass(cpt)
    ftab = sc_pass(srcp, dstp, ltab, ltab, rtab, ztab, ones)
    hh = sc_pass(srcp, dstp, rtab, rtab, ftab, xtab, m0)
    return jnp.concatenate([hh[:n], hh[NPAD:NPAD + n]], axis=1)
